# Initial kernel scaffold; baseline (speedup 1.0000x reference)
import jax
import jax.numpy as jnp
from jax.experimental import pallas as pl


def kernel(attn_s):
    raise NotImplementedError("write your pallas kernel here")


# speedup vs baseline: 92.2170x; 92.2170x over previous
import functools

import jax
import jax.numpy as jnp
from jax import lax
from jax.experimental import pallas as pl
from jax.experimental.pallas import tpu as pltpu

_TOPK = 512
_N = 1000000
_ROWS = 977
_COLS = 1024
_NPAD = _ROWS * _COLS


def _body(x_ref, o_ref, key_ref):
    x = x_ref[...]
    mx = jnp.max(x)

    b = lax.bitcast_convert_type(x, jnp.int32)
    key_s = jnp.where(b < 0, jnp.bitwise_xor(b, jnp.int32(0x7FFFFFFF)), b)
    key = lax.bitcast_convert_type(key_s, jnp.uint32) ^ jnp.uint32(0x80000000)
    key_ref[...] = key

    def step(i, t):
        cand = jnp.bitwise_or(t, jnp.uint32(1) << (jnp.uint32(31) - i.astype(jnp.uint32)))
        cnt = jnp.sum((key_ref[...] >= cand).astype(jnp.int32))
        return jnp.where(cnt >= _TOPK, cand, t)

    t = lax.fori_loop(0, 32, step, jnp.uint32(0))

    ts = lax.bitcast_convert_type(t ^ jnp.uint32(0x80000000), jnp.int32)
    db = jnp.where(ts < 0, jnp.bitwise_xor(ts, jnp.int32(0x7FFFFFFF)), ts)
    delta = lax.bitcast_convert_type(db, jnp.float32)

    w = jnp.maximum(x - mx + delta, 0.0)
    s = jnp.sum(w) + jnp.float32(1e-7)
    o_ref[...] = w * (1.0 / s)


@jax.jit
def kernel(attn_s):
    x = jnp.pad(attn_s, ((0, 0), (0, _NPAD - _N)), constant_values=-jnp.inf)
    x = x.reshape(_ROWS, _COLS)
    out = pl.pallas_call(
        _body,
        out_shape=jax.ShapeDtypeStruct((_ROWS, _COLS), jnp.float32),
        scratch_shapes=[pltpu.VMEM((_ROWS, _COLS), jnp.uint32)],
    )(x)
    return out.reshape(1, _NPAD)[:, :_N]
